# Initial kernel scaffold; baseline (speedup 1.0000x reference)
import jax
import jax.numpy as jnp
from jax.experimental import pallas as pl


def kernel(x, edge_index, emb, W1, b1, gamma1, beta1, W2, b2, gamma2, beta2, Wm1, bm1, Wm2, bm2):
    raise NotImplementedError("write your pallas kernel here")


# speedup vs baseline: 32.0647x; 32.0647x over previous
import functools

import jax
import jax.numpy as jnp
from jax import lax
from jax.experimental import pallas as pl
from jax.experimental.pallas import tpu as pltpu
from jax.experimental.pallas import tpu_sc as plsc

N = 100000
E = 3200000
D = 25
VOCAB = 10000

DP = 32
HC = 16
NC, NS = 2, 16

NP = 100352
PT = NP // NS

EPAD = 3211264
EW = EPAD // 32
ET = EPAD // NS
EROWS = EPAD // 128

NX = 102400

_f32 = jnp.float32
_zeros16 = lambda: jnp.zeros((16,), _f32)


def _sc_deg_emb_body(dst2, xp, emb1p, degp, h0, acc, eidx, ones, xidx, erows,
                     zbuf, sem):
    c = lax.axis_index("c")
    s = lax.axis_index("s")
    w = s * NC + c

    @pl.loop(0, PT // 16)
    def _z2(i):
        zbuf[pl.ds(i * 16, 16)] = _zeros16()

    @pl.loop(0, 8)
    def _o(i):
        ones[pl.ds(i * 16, 16)] = jnp.full((16,), 1.0, _f32)

    pltpu.sync_copy(zbuf, acc.at[pl.ds(s * PT, PT)])
    plsc.subcore_barrier()

    xbase = w * 3200
    pltpu.sync_copy(xp.at[pl.ds(xbase, 3200)], xidx)

    @pl.loop(0, 25)
    def _emb(j):
        pltpu.async_copy(emb1p.at[xidx.at[pl.ds(j * 128, 128)]], erows,
                         sem).wait()
        pltpu.sync_copy(erows, h0.at[pl.ds(xbase + j * 128, 128), :])

    rowbase = w * (EW // 128)

    @pl.loop(0, EW // 2048)
    def _deg(blk):
        pltpu.sync_copy(dst2.at[pl.ds(rowbase + blk * 16, 16), :], eidx)
        for j in range(16):
            pltpu.sync_copy(ones, acc.at[eidx.at[j]], add=True)

    plsc.subcore_barrier()
    pltpu.sync_copy(acc.at[pl.ds(s * PT, PT)], degp.at[c, pl.ds(s * PT, PT)])


_KB = 8


def _sc_msg_body(src2, dst2, gb, m, acc, sidx, didx, rows, sem):
    c = lax.axis_index("c")
    s = lax.axis_index("s")

    @pl.loop(0, _KB * 128)
    def _z1(i):
        rows[i] = _zeros16()

    @pl.loop(0, PT // (_KB * 128))
    def _z2(q):
        pltpu.sync_copy(rows, acc.at[pl.ds(s * PT + q * _KB * 128, _KB * 128), :])

    pltpu.sync_copy(rows.at[pl.ds(0, PT % (_KB * 128)), :],
                    acc.at[pl.ds(s * PT + PT - PT % (_KB * 128),
                                 PT % (_KB * 128)), :])
    plsc.subcore_barrier()

    rowbase = s * (ET // 128)

    @pl.loop(0, ET // (128 * _KB))
    def _blk(b):
        pltpu.sync_copy(src2.at[c, pl.ds(rowbase + b * _KB, _KB), :], sidx)
        pltpu.sync_copy(dst2.at[pl.ds(rowbase + b * _KB, _KB), :], didx)
        descs = [
            pltpu.async_copy(gb.at[sidx.at[j]],
                             rows.at[pl.ds(j * 128, 128), :], sem)
            for j in range(_KB)
        ]
        for d in descs:
            d.wait()
        for j in range(_KB):
            pltpu.sync_copy(rows.at[pl.ds(j * 128, 128), :],
                            acc.at[didx.at[j]], add=True)

    plsc.subcore_barrier()
    pltpu.sync_copy(acc.at[pl.ds(s * PT, PT), :],
                    m.at[c, pl.ds(s * PT, PT), :])


@functools.lru_cache(maxsize=1)
def _sc_kernels():
    mesh = plsc.VectorSubcoreMesh(core_axis_name="c", subcore_axis_name="s",
                                  num_cores=NC, num_subcores=NS)
    params = pltpu.CompilerParams(use_tc_tiling_on_sc=False)
    deg_emb = pl.kernel(
        _sc_deg_emb_body,
        out_type=(jax.ShapeDtypeStruct((NC, NP), _f32),
                  jax.ShapeDtypeStruct((NX, DP), _f32)),
        mesh=mesh,
        scratch_types=[
            pltpu.VMEM_SHARED((NP,), _f32),
            pltpu.VMEM((16, 128), jnp.int32),
            pltpu.VMEM((128,), _f32),
            pltpu.VMEM((3200,), jnp.int32),
            pltpu.VMEM((128, DP), _f32),
            pltpu.VMEM((PT,), _f32),
            pltpu.SemaphoreType.DMA,
        ],
        compiler_params=params,
    )
    msg = pl.kernel(
        _sc_msg_body,
        out_type=jax.ShapeDtypeStruct((NC, NP, HC), _f32),
        mesh=mesh,
        scratch_types=[
            pltpu.VMEM_SHARED((NP, HC), _f32),
            pltpu.VMEM((_KB, 128), jnp.int32),
            pltpu.VMEM((_KB, 128), jnp.int32),
            pltpu.VMEM((_KB * 128, HC), _f32),
            pltpu.SemaphoreType.DMA,
        ],
        compiler_params=params,
    )
    return deg_emb, msg


_RB = 2048
_GRID = (N + _RB - 1) // _RB


def _p0_body(emb_ref, w_ref, o_ref):
    o_ref[...] = jnp.dot(emb_ref[...], w_ref[...],
                         preferred_element_type=_f32)


def _p1_body(degp_ref, h0_ref, dis_ref, gb_ref):
    i = pl.program_id(0)
    dp = degp_ref[:, pl.ds(i * _RB, _RB)]
    deg = dp[0] + dp[1] + 1.0
    d = lax.rsqrt(deg)[:, None]
    dis_ref[...] = d
    g = h0_ref[...] * d
    gb_ref[0] = g[:, :HC]
    gb_ref[1] = g[:, HC:]


def _p2a_body(m_ref, g_ref, dis_ref, b_ref, conv_ref, ssum_ref, ssq_ref):
    i = pl.program_id(0)
    mm = m_ref[...]
    gg = g_ref[...]
    msum = jnp.concatenate([mm[0], mm[1]], axis=1)
    g = jnp.concatenate([gg[0], gg[1]], axis=1)
    conv = dis_ref[...] * (msum + g) + b_ref[...]
    conv_ref[...] = conv
    row = i * _RB + lax.broadcasted_iota(jnp.int32, (_RB, 1), 0)
    cm = jnp.where(row < N, conv, 0.0)
    ps = jnp.sum(cm, axis=0, keepdims=True)
    pq = jnp.sum(cm * cm, axis=0, keepdims=True)

    @pl.when(i == 0)
    def _():
        ssum_ref[...] = ps
        ssq_ref[...] = pq

    @pl.when(i > 0)
    def _():
        ssum_ref[...] += ps
        ssq_ref[...] += pq


def _bn_relu(conv, ssum, ssq, gamma, beta):
    mu = ssum / N
    var = ssq / N - mu * mu
    r = gamma * lax.rsqrt(var + 1e-5)
    return jnp.maximum(conv * r + (beta - mu * r), 0.0)


def _p2b_body(conv_ref, ssum_ref, ssq_ref, gam_ref, bet_ref, w_ref, dis_ref,
              gb_ref):
    h = _bn_relu(conv_ref[...], ssum_ref[...], ssq_ref[...], gam_ref[...],
                 bet_ref[...])
    g = jnp.dot(h, w_ref[...], preferred_element_type=_f32) * dis_ref[...]
    gb_ref[0] = g[:, :HC]
    gb_ref[1] = g[:, HC:]


def _p3_body(conv_ref, ssum_ref, ssq_ref, gam_ref, bet_ref, wm1_ref, bm1_ref,
             wm2_ref, bm2_ref, out_ref):
    h = _bn_relu(conv_ref[...], ssum_ref[...], ssq_ref[...], gam_ref[...],
                 bet_ref[...])
    t = jnp.maximum(
        jnp.dot(h, wm1_ref[...], preferred_element_type=_f32) + bm1_ref[...],
        0.0)
    o = jnp.dot(t, wm2_ref[...], preferred_element_type=_f32) + bm2_ref[...]
    out_ref[...] = jax.nn.sigmoid(o[:, 0:1])


def _blk2(shape, imap):
    return pl.BlockSpec(shape, imap)


_im_row = lambda i: (i, 0)
_im_3d = lambda i: (0, i, 0)
_im_full2 = lambda i: (0, 0)


def kernel(x, edge_index, emb, W1, b1, gamma1, beta1, W2, b2, gamma2, beta2,
           Wm1, bm1, Wm2, bm2):
    f32 = _f32
    i32 = jnp.int32

    src = edge_index[0]
    dst = edge_index[1]
    src_p = jnp.concatenate([src, jnp.zeros((EPAD - E,), i32)])
    dst_p = jnp.concatenate([dst, jnp.full((EPAD - E,), N, i32)])
    src2 = jnp.stack([src_p, src_p + NP]).reshape(NC, EROWS, 128)
    dst2d = dst_p.reshape(EROWS, 128)
    xp = jnp.concatenate([x, jnp.zeros((NX - N,), i32)])

    embp = jnp.pad(emb, ((0, 0), (0, DP - D)))
    W1p = jnp.pad(W1, ((0, DP - D), (0, DP - D)))
    W2p = jnp.pad(W2, ((0, DP - D), (0, DP - D)))
    b1p = jnp.pad(b1, (0, DP - D)).reshape(1, DP)
    b2p = jnp.pad(b2, (0, DP - D)).reshape(1, DP)
    g1p = jnp.pad(gamma1, (0, DP - D)).reshape(1, DP)
    g2p = jnp.pad(gamma2, (0, DP - D)).reshape(1, DP)
    be1p = jnp.pad(beta1, (0, DP - D)).reshape(1, DP)
    be2p = jnp.pad(beta2, (0, DP - D)).reshape(1, DP)
    Wm1p = jnp.pad(Wm1, ((0, DP - D), (0, 16 - 12)))
    bm1p = jnp.pad(bm1, (0, 16 - 12)).reshape(1, 16)
    Wm2p = jnp.pad(Wm2, ((0, 16 - 12), (0, 8 - 1)))
    bm2p = jnp.pad(bm2, (0, 8 - 1)).reshape(1, 8)

    emb1p = pl.pallas_call(
        _p0_body,
        out_shape=jax.ShapeDtypeStruct((VOCAB, DP), f32),
    )(embp, W1p)

    _sc_deg_emb, _sc_msg = _sc_kernels()
    degp, h0 = _sc_deg_emb(dst2d, xp, emb1p)

    dis, gb1 = pl.pallas_call(
        _p1_body,
        grid=(_GRID,),
        in_specs=[
            _blk2((NC, NP), lambda i: (0, 0)),
            _blk2((_RB, DP), _im_row),
        ],
        out_specs=[
            _blk2((_RB, 1), _im_row),
            _blk2((NC, _RB, HC), _im_3d),
        ],
        out_shape=[
            jax.ShapeDtypeStruct((N, 1), f32),
            jax.ShapeDtypeStruct((NC, NP, HC), f32),
        ],
    )(degp, h0)

    msg_in_specs = [
        _blk2((NC, _RB, HC), _im_3d),
        _blk2((NC, _RB, HC), _im_3d),
        _blk2((_RB, 1), _im_row),
        _blk2((1, DP), _im_full2),
    ]
    msg_out_specs = [
        _blk2((_RB, DP), _im_row),
        _blk2((1, DP), _im_full2),
        _blk2((1, DP), _im_full2),
    ]
    msg_out_shape = [
        jax.ShapeDtypeStruct((N, DP), f32),
        jax.ShapeDtypeStruct((1, DP), f32),
        jax.ShapeDtypeStruct((1, DP), f32),
    ]

    m1 = _sc_msg(src2, dst2d, gb1.reshape(NC * NP, HC))
    conv1, s1, q1 = pl.pallas_call(
        _p2a_body, grid=(_GRID,), in_specs=msg_in_specs,
        out_specs=msg_out_specs, out_shape=msg_out_shape,
    )(m1, gb1, dis, b1p)

    gb2 = pl.pallas_call(
        _p2b_body,
        grid=(_GRID,),
        in_specs=[
            _blk2((_RB, DP), _im_row),
            _blk2((1, DP), _im_full2),
            _blk2((1, DP), _im_full2),
            _blk2((1, DP), _im_full2),
            _blk2((1, DP), _im_full2),
            _blk2((DP, DP), _im_full2),
            _blk2((_RB, 1), _im_row),
        ],
        out_specs=_blk2((NC, _RB, HC), _im_3d),
        out_shape=jax.ShapeDtypeStruct((NC, NP, HC), f32),
    )(conv1, s1, q1, g1p, be1p, W2p, dis)

    m2 = _sc_msg(src2, dst2d, gb2.reshape(NC * NP, HC))
    conv2, s2, q2 = pl.pallas_call(
        _p2a_body, grid=(_GRID,), in_specs=msg_in_specs,
        out_specs=msg_out_specs, out_shape=msg_out_shape,
    )(m2, gb2, dis, b2p)

    out = pl.pallas_call(
        _p3_body,
        grid=(_GRID,),
        in_specs=[
            _blk2((_RB, DP), _im_row),
            _blk2((1, DP), _im_full2),
            _blk2((1, DP), _im_full2),
            _blk2((1, DP), _im_full2),
            _blk2((1, DP), _im_full2),
            _blk2((DP, 16), _im_full2),
            _blk2((1, 16), _im_full2),
            _blk2((16, 8), _im_full2),
            _blk2((1, 8), _im_full2),
        ],
        out_specs=_blk2((_RB, 1), _im_row),
        out_shape=jax.ShapeDtypeStruct((N, 1), f32),
    )(conv2, s2, q2, g2p, be2p, Wm1p, bm1p, Wm2p, bm2p)

    return out


# speedup vs baseline: 36.7535x; 1.1462x over previous
import functools

import jax
import jax.numpy as jnp
from jax import lax
from jax.experimental import pallas as pl
from jax.experimental.pallas import tpu as pltpu
from jax.experimental.pallas import tpu_sc as plsc

N = 100000
E = 3200000
D = 25
VOCAB = 10000

DP = 32
HC = 16
NC, NS = 2, 16

NP = 100352
PT = NP // NS

EPAD = 3211264
EW = EPAD // 32
ET = EPAD // NS
EROWS = EPAD // 128

NX = 102400

_f32 = jnp.float32
_zeros16 = lambda: jnp.zeros((16,), _f32)


def _sc_deg_emb_body(dst2, xp, emb1p, degp, h0, acc, eidx, ones, xidx, erows,
                     zbuf, sem):
    c = lax.axis_index("c")
    s = lax.axis_index("s")
    w = s * NC + c

    @pl.loop(0, PT // 16)
    def _z2(i):
        zbuf[pl.ds(i * 16, 16)] = _zeros16()

    @pl.loop(0, 8)
    def _o(i):
        ones[pl.ds(i * 16, 16)] = jnp.full((16,), 1.0, _f32)

    pltpu.sync_copy(zbuf, acc.at[pl.ds(s * PT, PT)])
    plsc.subcore_barrier()

    xbase = w * 3200
    pltpu.sync_copy(xp.at[pl.ds(xbase, 3200)], xidx)

    @pl.loop(0, 25)
    def _emb(j):
        pltpu.async_copy(emb1p.at[xidx.at[pl.ds(j * 128, 128)]], erows,
                         sem).wait()
        pltpu.sync_copy(erows, h0.at[pl.ds(xbase + j * 128, 128), :])

    rowbase = w * (EW // 128)

    @pl.loop(0, EW // 2048)
    def _deg(blk):
        pltpu.sync_copy(dst2.at[pl.ds(rowbase + blk * 16, 16), :], eidx)
        for j in range(16):
            pltpu.sync_copy(ones, acc.at[eidx.at[j]], add=True)

    plsc.subcore_barrier()
    pltpu.sync_copy(acc.at[pl.ds(s * PT, PT)], degp.at[c, pl.ds(s * PT, PT)])


_KB = 4
_BW = _KB * 128
_NBLK = ET // _BW
_NPAIR = _NBLK // 2


def _sc_msg_body(src2, dst2, gb, m, acc, sidxa, sidxb, didxa, didxb, rowsa,
                 rowsb, sem):
    c = lax.axis_index("c")
    s = lax.axis_index("s")

    @pl.loop(0, _BW)
    def _z1(i):
        rowsa[i] = _zeros16()

    @pl.loop(0, PT // _BW)
    def _z2(q):
        pltpu.sync_copy(rowsa, acc.at[pl.ds(s * PT + q * _BW, _BW), :])

    pltpu.sync_copy(rowsa.at[pl.ds(0, PT % _BW), :],
                    acc.at[pl.ds(s * PT + PT - PT % _BW, PT % _BW), :])
    plsc.subcore_barrier()

    rowbase = s * (ET // 128)
    sidx = (sidxa, sidxb)
    didx = (didxa, didxb)
    rows = (rowsa, rowsb)

    def load_and_fire(b, u):
        pltpu.sync_copy(src2.at[c, pl.ds(rowbase + b * _KB, _KB), :], sidx[u])
        pltpu.sync_copy(dst2.at[pl.ds(rowbase + b * _KB, _KB), :], didx[u])
        for j in range(_KB):
            pltpu.async_copy(gb.at[sidx[u].at[j]],
                             rows[u].at[pl.ds(j * 128, 128), :], sem)

    def drain_gathers(u):
        pltpu.make_async_copy(gb.at[pl.ds(0, _BW), :], rows[u], sem).wait()

    load_and_fire(0, 0)

    @pl.loop(0, _NPAIR)
    def _pair(k):
        for u in range(2):
            b = 2 * k + u

            @pl.when(b + 1 < _NBLK)
            def _():
                load_and_fire(b + 1, 1 - u)

            drain_gathers(u)
            for j in range(_KB):
                pltpu.sync_copy(rows[u].at[pl.ds(j * 128, 128), :],
                                acc.at[didx[u].at[j]], add=True)

    plsc.subcore_barrier()
    pltpu.sync_copy(acc.at[pl.ds(s * PT, PT), :],
                    m.at[c, pl.ds(s * PT, PT), :])


@functools.lru_cache(maxsize=1)
def _sc_kernels():
    mesh = plsc.VectorSubcoreMesh(core_axis_name="c", subcore_axis_name="s",
                                  num_cores=NC, num_subcores=NS)
    params = pltpu.CompilerParams(use_tc_tiling_on_sc=False)
    deg_emb = pl.kernel(
        _sc_deg_emb_body,
        out_type=(jax.ShapeDtypeStruct((NC, NP), _f32),
                  jax.ShapeDtypeStruct((NX, DP), _f32)),
        mesh=mesh,
        scratch_types=[
            pltpu.VMEM_SHARED((NP,), _f32),
            pltpu.VMEM((16, 128), jnp.int32),
            pltpu.VMEM((128,), _f32),
            pltpu.VMEM((3200,), jnp.int32),
            pltpu.VMEM((128, DP), _f32),
            pltpu.VMEM((PT,), _f32),
            pltpu.SemaphoreType.DMA,
        ],
        compiler_params=params,
    )
    msg = pl.kernel(
        _sc_msg_body,
        out_type=jax.ShapeDtypeStruct((NC, NP, HC), _f32),
        mesh=mesh,
        scratch_types=[
            pltpu.VMEM_SHARED((NP, HC), _f32),
            pltpu.VMEM((_KB, 128), jnp.int32),
            pltpu.VMEM((_KB, 128), jnp.int32),
            pltpu.VMEM((_KB, 128), jnp.int32),
            pltpu.VMEM((_KB, 128), jnp.int32),
            pltpu.VMEM((_BW, HC), _f32),
            pltpu.VMEM((_BW, HC), _f32),
            pltpu.SemaphoreType.DMA,
        ],
        compiler_params=params,
    )
    return deg_emb, msg


_RB = 2048
_GRID = (N + _RB - 1) // _RB


def _p0_body(emb_ref, w_ref, o_ref):
    o_ref[...] = jnp.dot(emb_ref[...], w_ref[...],
                         preferred_element_type=_f32)


def _p1_body(degp_ref, h0_ref, dis_ref, gb_ref):
    i = pl.program_id(0)
    dp = degp_ref[:, pl.ds(i * _RB, _RB)]
    deg = dp[0] + dp[1] + 1.0
    d = lax.rsqrt(deg)[:, None]
    dis_ref[...] = d
    g = h0_ref[...] * d
    gb_ref[0] = g[:, :HC]
    gb_ref[1] = g[:, HC:]


def _p2a_body(m_ref, g_ref, dis_ref, b_ref, conv_ref, ssum_ref, ssq_ref):
    i = pl.program_id(0)
    mm = m_ref[...]
    gg = g_ref[...]
    msum = jnp.concatenate([mm[0], mm[1]], axis=1)
    g = jnp.concatenate([gg[0], gg[1]], axis=1)
    conv = dis_ref[...] * (msum + g) + b_ref[...]
    conv_ref[...] = conv
    row = i * _RB + lax.broadcasted_iota(jnp.int32, (_RB, 1), 0)
    cm = jnp.where(row < N, conv, 0.0)
    ps = jnp.sum(cm, axis=0, keepdims=True)
    pq = jnp.sum(cm * cm, axis=0, keepdims=True)

    @pl.when(i == 0)
    def _():
        ssum_ref[...] = ps
        ssq_ref[...] = pq

    @pl.when(i > 0)
    def _():
        ssum_ref[...] += ps
        ssq_ref[...] += pq


def _bn_relu(conv, ssum, ssq, gamma, beta):
    mu = ssum / N
    var = ssq / N - mu * mu
    r = gamma * lax.rsqrt(var + 1e-5)
    return jnp.maximum(conv * r + (beta - mu * r), 0.0)


def _p2b_body(conv_ref, ssum_ref, ssq_ref, gam_ref, bet_ref, w_ref, dis_ref,
              gb_ref):
    h = _bn_relu(conv_ref[...], ssum_ref[...], ssq_ref[...], gam_ref[...],
                 bet_ref[...])
    g = jnp.dot(h, w_ref[...], preferred_element_type=_f32) * dis_ref[...]
    gb_ref[0] = g[:, :HC]
    gb_ref[1] = g[:, HC:]


def _p3_body(conv_ref, ssum_ref, ssq_ref, gam_ref, bet_ref, wm1_ref, bm1_ref,
             wm2_ref, bm2_ref, out_ref):
    h = _bn_relu(conv_ref[...], ssum_ref[...], ssq_ref[...], gam_ref[...],
                 bet_ref[...])
    t = jnp.maximum(
        jnp.dot(h, wm1_ref[...], preferred_element_type=_f32) + bm1_ref[...],
        0.0)
    o = jnp.dot(t, wm2_ref[...], preferred_element_type=_f32) + bm2_ref[...]
    out_ref[...] = jax.nn.sigmoid(o[:, 0:1])


def _blk2(shape, imap):
    return pl.BlockSpec(shape, imap)


_im_row = lambda i: (i, 0)
_im_3d = lambda i: (0, i, 0)
_im_full2 = lambda i: (0, 0)


def kernel(x, edge_index, emb, W1, b1, gamma1, beta1, W2, b2, gamma2, beta2,
           Wm1, bm1, Wm2, bm2):
    f32 = _f32
    i32 = jnp.int32

    src = edge_index[0]
    dst = edge_index[1]
    src_p = jnp.concatenate([src, jnp.zeros((EPAD - E,), i32)])
    dst_p = jnp.concatenate([dst, jnp.full((EPAD - E,), N, i32)])
    src2 = jnp.stack([src_p, src_p + NP]).reshape(NC, EROWS, 128)
    dst2d = dst_p.reshape(EROWS, 128)
    xp = jnp.concatenate([x, jnp.zeros((NX - N,), i32)])

    embp = jnp.pad(emb, ((0, 0), (0, DP - D)))
    W1p = jnp.pad(W1, ((0, DP - D), (0, DP - D)))
    W2p = jnp.pad(W2, ((0, DP - D), (0, DP - D)))
    b1p = jnp.pad(b1, (0, DP - D)).reshape(1, DP)
    b2p = jnp.pad(b2, (0, DP - D)).reshape(1, DP)
    g1p = jnp.pad(gamma1, (0, DP - D)).reshape(1, DP)
    g2p = jnp.pad(gamma2, (0, DP - D)).reshape(1, DP)
    be1p = jnp.pad(beta1, (0, DP - D)).reshape(1, DP)
    be2p = jnp.pad(beta2, (0, DP - D)).reshape(1, DP)
    Wm1p = jnp.pad(Wm1, ((0, DP - D), (0, 16 - 12)))
    bm1p = jnp.pad(bm1, (0, 16 - 12)).reshape(1, 16)
    Wm2p = jnp.pad(Wm2, ((0, 16 - 12), (0, 8 - 1)))
    bm2p = jnp.pad(bm2, (0, 8 - 1)).reshape(1, 8)

    emb1p = pl.pallas_call(
        _p0_body,
        out_shape=jax.ShapeDtypeStruct((VOCAB, DP), f32),
    )(embp, W1p)

    _sc_deg_emb, _sc_msg = _sc_kernels()
    degp, h0 = _sc_deg_emb(dst2d, xp, emb1p)

    dis, gb1 = pl.pallas_call(
        _p1_body,
        grid=(_GRID,),
        in_specs=[
            _blk2((NC, NP), lambda i: (0, 0)),
            _blk2((_RB, DP), _im_row),
        ],
        out_specs=[
            _blk2((_RB, 1), _im_row),
            _blk2((NC, _RB, HC), _im_3d),
        ],
        out_shape=[
            jax.ShapeDtypeStruct((N, 1), f32),
            jax.ShapeDtypeStruct((NC, NP, HC), f32),
        ],
    )(degp, h0)

    msg_in_specs = [
        _blk2((NC, _RB, HC), _im_3d),
        _blk2((NC, _RB, HC), _im_3d),
        _blk2((_RB, 1), _im_row),
        _blk2((1, DP), _im_full2),
    ]
    msg_out_specs = [
        _blk2((_RB, DP), _im_row),
        _blk2((1, DP), _im_full2),
        _blk2((1, DP), _im_full2),
    ]
    msg_out_shape = [
        jax.ShapeDtypeStruct((N, DP), f32),
        jax.ShapeDtypeStruct((1, DP), f32),
        jax.ShapeDtypeStruct((1, DP), f32),
    ]

    m1 = _sc_msg(src2, dst2d, gb1.reshape(NC * NP, HC))
    conv1, s1, q1 = pl.pallas_call(
        _p2a_body, grid=(_GRID,), in_specs=msg_in_specs,
        out_specs=msg_out_specs, out_shape=msg_out_shape,
    )(m1, gb1, dis, b1p)

    gb2 = pl.pallas_call(
        _p2b_body,
        grid=(_GRID,),
        in_specs=[
            _blk2((_RB, DP), _im_row),
            _blk2((1, DP), _im_full2),
            _blk2((1, DP), _im_full2),
            _blk2((1, DP), _im_full2),
            _blk2((1, DP), _im_full2),
            _blk2((DP, DP), _im_full2),
            _blk2((_RB, 1), _im_row),
        ],
        out_specs=_blk2((NC, _RB, HC), _im_3d),
        out_shape=jax.ShapeDtypeStruct((NC, NP, HC), f32),
    )(conv1, s1, q1, g1p, be1p, W2p, dis)

    m2 = _sc_msg(src2, dst2d, gb2.reshape(NC * NP, HC))
    conv2, s2, q2 = pl.pallas_call(
        _p2a_body, grid=(_GRID,), in_specs=msg_in_specs,
        out_specs=msg_out_specs, out_shape=msg_out_shape,
    )(m2, gb2, dis, b2p)

    out = pl.pallas_call(
        _p3_body,
        grid=(_GRID,),
        in_specs=[
            _blk2((_RB, DP), _im_row),
            _blk2((1, DP), _im_full2),
            _blk2((1, DP), _im_full2),
            _blk2((1, DP), _im_full2),
            _blk2((1, DP), _im_full2),
            _blk2((DP, 16), _im_full2),
            _blk2((1, 16), _im_full2),
            _blk2((16, 8), _im_full2),
            _blk2((1, 8), _im_full2),
        ],
        out_specs=_blk2((_RB, 1), _im_row),
        out_shape=jax.ShapeDtypeStruct((N, 1), f32),
    )(conv2, s2, q2, g2p, be2p, Wm1p, bm1p, Wm2p, bm2p)

    return out
